# Initial kernel scaffold; baseline (speedup 1.0000x reference)
import jax
import jax.numpy as jnp
from jax.experimental import pallas as pl


def kernel(x, W_enc, W_base, Q, J_raw, b_enc, b_dec):
    raise NotImplementedError("write your pallas kernel here")


# speedup vs baseline: 11.5807x; 11.5807x over previous
import functools

import jax
import jax.numpy as jnp
from jax import lax
from jax.experimental import pallas as pl

K_TOP = 64


def _encode_kernel(x_ref, w_ref, b_ref, out_ref):
    out_ref[...] = (
        jnp.dot(x_ref[...], w_ref[...], preferred_element_type=jnp.float32)
        + b_ref[...]
    )


def _encode(x2, w2, b):
    B, K = x2.shape
    S = w2.shape[1]
    BN = 512
    grid = (S // BN,)
    return pl.pallas_call(
        _encode_kernel,
        grid=grid,
        in_specs=[
            pl.BlockSpec((B, K), lambda n: (0, 0)),
            pl.BlockSpec((K, BN), lambda n: (0, n)),
            pl.BlockSpec((1, BN), lambda n: (0, n)),
        ],
        out_specs=pl.BlockSpec((B, BN), lambda n: (0, n)),
        out_shape=jax.ShapeDtypeStruct((B, S), jnp.float32),
    )(x2, w2, b)


def _sparse_code_kernel(pre_ref, z_ref):
    pre = pre_ref[...]
    i = lax.bitcast_convert_type(pre, jnp.int32)
    key = jnp.where(i >= 0, i, i ^ jnp.int32(0x7FFFFFFF))
    lo = jnp.min(key, axis=1, keepdims=True)
    hi = jnp.max(key, axis=1, keepdims=True)

    def body(_, carry):
        lo, hi = carry
        mid = (lo >> 1) + (hi >> 1) + (lo & hi & 1)
        cnt = jnp.sum((key >= mid).astype(jnp.int32), axis=1, keepdims=True)
        ge = cnt >= K_TOP
        return jnp.where(ge, mid, lo), jnp.where(ge, hi, mid)

    lo, hi = lax.fori_loop(0, 32, body, (lo, hi))
    mask = key >= lo
    z_ref[...] = jnp.where(mask, jnp.maximum(pre, 0.0), 0.0)


def _sparse_code(pre):
    B, S = pre.shape
    BM = 128
    return pl.pallas_call(
        _sparse_code_kernel,
        grid=(B // BM,),
        in_specs=[pl.BlockSpec((BM, S), lambda m: (m, 0))],
        out_specs=pl.BlockSpec((BM, S), lambda m: (m, 0)),
        out_shape=jax.ShapeDtypeStruct((B, S), jnp.float32),
    )(pre)


def _decode_kernel(z_ref, w_ref, out_ref):
    part = jnp.dot(z_ref[...], w_ref[...], preferred_element_type=jnp.float32)

    @pl.when(pl.program_id(0) == 0)
    def _():
        out_ref[...] = part

    @pl.when(pl.program_id(0) != 0)
    def _():
        out_ref[...] += part


def _decode(z, w_base):
    B, S = z.shape
    D = w_base.shape[1]
    KB = 2048
    return pl.pallas_call(
        _decode_kernel,
        grid=(S // KB,),
        in_specs=[
            pl.BlockSpec((B, KB), lambda k: (0, k)),
            pl.BlockSpec((KB, D), lambda k: (k, 0)),
        ],
        out_specs=pl.BlockSpec((B, D), lambda k: (0, 0)),
        out_shape=jax.ShapeDtypeStruct((B, D), jnp.float32),
    )(z, w_base)


def _assemble_kernel(base_ref, q_ref, qt_ref, j_ref, bdec_ref, x_ref,
                     xhat_ref, loss_ref, *, T, D):
    base = base_ref[...]
    proj = jnp.dot(base, q_ref[...], preferred_element_type=jnp.float32)

    jraw = j_ref[...]
    A = (jraw - jraw.T) * 0.5
    A1 = A * (1.0 / 64.0)
    r = lax.broadcasted_iota(jnp.int32, (8, 8), 0)
    c = lax.broadcasted_iota(jnp.int32, (8, 8), 1)
    I8 = (r == c).astype(jnp.float32)

    term = I8
    E = I8
    for i in range(1, 19):
        term = jnp.dot(term, A1, preferred_element_type=jnp.float32) / float(i)
        E = E + term
    for _ in range(6):
        E = jnp.dot(E, E, preferred_element_type=jnp.float32)
    E1 = E
    E2 = jnp.dot(E1, E1, preferred_element_type=jnp.float32)
    E3 = jnp.dot(E2, E1, preferred_element_type=jnp.float32)

    qt = qt_ref[...]
    total = None
    for t, Et in ((0, None), (1, E1), (2, E2), (3, E3)):
        xh = base + bdec_ref[pl.ds(t, 1), :]
        if Et is not None:
            G = jnp.dot(Et - I8, qt, preferred_element_type=jnp.float32)
            xh = xh + jnp.dot(proj, G, preferred_element_type=jnp.float32)
        xhat_ref[:, t * D:(t + 1) * D] = xh
        d = xh - x_ref[:, t * D:(t + 1) * D]
        s = jnp.sum(d * d, axis=(0, 1), keepdims=True)
        total = s if total is None else total + s
    loss_ref[...] = total * (1.0 / (base.shape[0] * T))


def _assemble(base, q, qt, j_raw, b_dec, x2):
    B, D = base.shape
    T = b_dec.shape[0]
    return pl.pallas_call(
        functools.partial(_assemble_kernel, T=T, D=D),
        in_specs=[
            pl.BlockSpec(base.shape, lambda: (0, 0)),
            pl.BlockSpec(q.shape, lambda: (0, 0)),
            pl.BlockSpec(qt.shape, lambda: (0, 0)),
            pl.BlockSpec(j_raw.shape, lambda: (0, 0)),
            pl.BlockSpec(b_dec.shape, lambda: (0, 0)),
            pl.BlockSpec(x2.shape, lambda: (0, 0)),
        ],
        out_specs=[
            pl.BlockSpec((B, T * D), lambda: (0, 0)),
            pl.BlockSpec((1, 1), lambda: (0, 0)),
        ],
        out_shape=[
            jax.ShapeDtypeStruct((B, T * D), jnp.float32),
            jax.ShapeDtypeStruct((1, 1), jnp.float32),
        ],
    )(base, q, qt, j_raw, b_dec, x2)


def kernel(x, W_enc, W_base, Q, J_raw, b_enc, b_dec):
    B, T, D = x.shape
    S = W_enc.shape[2]
    x2 = x.reshape(B, T * D)
    w2 = W_enc.reshape(T * D, S)
    pre = _encode(x2, w2, b_enc.reshape(1, S))
    z = _sparse_code(pre)
    base = _decode(z, W_base)
    xhat2, loss = _assemble(base, Q, Q.T, J_raw, b_dec, x2)
    return (loss[0, 0], xhat2.reshape(B, T, D), z)
